# baseline (speedup 1.0000x reference)
import functools

import jax
import jax.numpy as jnp
from jax import lax
from jax.experimental import pallas as pl
from jax.experimental.pallas import tpu as pltpu
from jax.experimental.pallas import tpu_sc as plsc

N = 10000
E = 320000
D = 128
DH = D // 2

NC = 2
NS = 16
CH = 128
NBUF = 4
CPT = 160
Q = 32
NGRP = CPT // Q
HBM_SLOTS = (3,)
E_PAD = CPT * NS * CH
N_PAD = 10112
ZR = N_PAD // NS

_sc_mesh = plsc.VectorSubcoreMesh(core_axis_name="c", subcore_axis_name="s")


@functools.partial(
    pl.kernel,
    out_type=jax.ShapeDtypeStruct((NC, N_PAD, DH), jnp.float32),
    mesh=_sc_mesh,
    compiler_params=pltpu.CompilerParams(use_tc_tiling_on_sc=False),
    scratch_types=[
        pltpu.VMEM((NBUF, CH, DH), jnp.float32),
        pltpu.VMEM((2, Q, CH), jnp.int32),
        pltpu.VMEM((2, Q, CH), jnp.int32),
        pltpu.VMEM_SHARED((N_PAD, DH), jnp.float32),
        pltpu.VMEM_SHARED((N_PAD, DH), jnp.float32),
        pltpu.SemaphoreType.DMA,
        pltpu.SemaphoreType.DMA,
        pltpu.SemaphoreType.DMA,
        pltpu.SemaphoreType.DMA,
        pltpu.SemaphoreType.DMA,
        pltpu.SemaphoreType.DMA,
        pltpu.SemaphoreType.DMA,
        pltpu.SemaphoreType.DMA,
        pltpu.SemaphoreType.DMA,
        pltpu.SemaphoreType.DMA,
    ],
)
def _sc_scatter(x_hbm, srcc_hbm, dstc_hbm, zeros_hbm, out_hbm,
                rows_v, srci_v, dsti_v, agg_sh, x_sh, *sems):
    cid = lax.axis_index("c")
    sid = lax.axis_index("s")
    sem_g = sems[:NBUF]
    sem_s = sems[NBUF:2 * NBUF]
    sem_i0, sem_i1 = sems[2 * NBUF:]

    pltpu.sync_copy(srcc_hbm.at[sid, pl.ds(0, Q)], srci_v.at[0])
    pltpu.sync_copy(dstc_hbm.at[sid, pl.ds(0, Q)], dsti_v.at[0])
    pltpu.sync_copy(x_hbm.at[cid, pl.ds(sid * ZR, ZR)], x_sh.at[pl.ds(sid * ZR, ZR)])
    pltpu.sync_copy(zeros_hbm, agg_sh.at[pl.ds(sid * ZR, ZR)])
    plsc.subcore_barrier()

    xh_hbm = x_hbm.at[cid]

    def gsrc(b):
        return xh_hbm if b in HBM_SLOTS else x_sh

    for grp in range(NGRP):
        buf = grp % 2
        if grp + 1 < NGRP:
            pltpu.async_copy(srcc_hbm.at[sid, pl.ds((grp + 1) * Q, Q)],
                             srci_v.at[1 - buf], sem_i0)
            pltpu.async_copy(dstc_hbm.at[sid, pl.ds((grp + 1) * Q, Q)],
                             dsti_v.at[1 - buf], sem_i1)

        sg = srci_v.at[buf]
        dg = dsti_v.at[buf]
        for b in range(2):
            pltpu.async_copy(gsrc(b).at[sg.at[b]], rows_v.at[b], sem_g[b])

        def group_fn(g, carry, sg=sg, dg=dg):
            for b in range(NBUF):
                k = g * NBUF + b
                b2 = (b + 2) % NBUF

                @pl.when(k + 2 >= NBUF)
                def _():
                    pltpu.make_async_copy(
                        rows_v.at[b2], agg_sh.at[dg.at[k - 2]], sem_s[b2]).wait()

                @pl.when(k + 2 < Q)
                def _():
                    pltpu.async_copy(gsrc(b2).at[sg.at[k + 2]], rows_v.at[b2], sem_g[b2])

                pltpu.make_async_copy(gsrc(b).at[sg.at[k]], rows_v.at[b], sem_g[b]).wait()
                pltpu.async_copy(rows_v.at[b], agg_sh.at[dg.at[k]], sem_s[b], add=True)
            return carry

        lax.fori_loop(0, Q // NBUF, group_fn, 0)
        for k in (Q - 2, Q - 1):
            b = k % NBUF
            pltpu.make_async_copy(rows_v.at[b], agg_sh.at[dg.at[k]], sem_s[b]).wait()
        if grp + 1 < NGRP:
            pltpu.make_async_copy(srcc_hbm.at[sid, pl.ds((grp + 1) * Q, Q)],
                                  srci_v.at[1 - buf], sem_i0).wait()
            pltpu.make_async_copy(dstc_hbm.at[sid, pl.ds((grp + 1) * Q, Q)],
                                  dsti_v.at[1 - buf], sem_i1).wait()

    plsc.subcore_barrier()
    pltpu.sync_copy(agg_sh.at[pl.ds(sid * ZR, ZR)],
                    out_hbm.at[cid, pl.ds(sid * ZR, ZR)])


def _tc_body(x_ref, w_ref, p_ref, o_ref):
    mm = jnp.dot(x_ref[...], w_ref[...], preferred_element_type=jnp.float32)
    agg = jnp.concatenate([p_ref[0], p_ref[1]], axis=1)
    o_ref[...] = jnp.maximum(agg + mm, 0.0)


_BLK = 1000


def _tc_finish(x, w, partials):
    grid = (N // _BLK,)
    return pl.pallas_call(
        _tc_body,
        grid=grid,
        in_specs=[
            pl.BlockSpec((_BLK, D), lambda i: (i, 0)),
            pl.BlockSpec((D, D), lambda i: (0, 0)),
            pl.BlockSpec((NC, _BLK, DH), lambda i: (0, i, 0)),
        ],
        out_specs=pl.BlockSpec((_BLK, D), lambda i: (i, 0)),
        out_shape=jax.ShapeDtypeStruct((N, D), jnp.float32),
    )(x, w, partials)


def kernel(x, edge_index, loop_weight):
    src = edge_index[0].astype(jnp.int32)
    dst = edge_index[1].astype(jnp.int32)
    pad = E_PAD - E
    src_c = jnp.concatenate([src, jnp.zeros((pad,), jnp.int32)]).reshape(NS, CPT, CH)
    dst_c = jnp.concatenate([dst, jnp.full((pad,), N, jnp.int32)]).reshape(NS, CPT, CH)
    zeros = jnp.zeros((ZR, DH), jnp.float32)
    x_split = x.reshape(N, NC, DH).transpose(1, 0, 2)
    x_split = jnp.concatenate(
        [x_split, jnp.zeros((NC, N_PAD - N, DH), jnp.float32)], axis=1)
    partials = _sc_scatter(x_split, src_c, dst_c, zeros)
    return _tc_finish(x, loop_weight, partials)


# speedup vs baseline: 1.2396x; 1.2396x over previous
import functools

import jax
import jax.numpy as jnp
from jax import lax
from jax.experimental import pallas as pl
from jax.experimental.pallas import tpu as pltpu
from jax.experimental.pallas import tpu_sc as plsc

N = 10000
E = 320000
D = 128
DH = D // 2

NC = 2
NS = 16
CH = 128
NBUF = 4
CPT = 160
Q = 32
NGRP = CPT // Q
HBM_SLOTS = ()
E_PAD = CPT * NS * CH
N_PAD = 10112
ZR = N_PAD // NS

_sc_mesh = plsc.VectorSubcoreMesh(core_axis_name="c", subcore_axis_name="s")


@functools.partial(
    pl.kernel,
    out_type=jax.ShapeDtypeStruct((NC, N_PAD, DH), jnp.float32),
    mesh=_sc_mesh,
    compiler_params=pltpu.CompilerParams(use_tc_tiling_on_sc=False),
    scratch_types=[
        pltpu.VMEM((NBUF, CH, DH), jnp.float32),
        pltpu.VMEM((2, Q, CH), jnp.int32),
        pltpu.VMEM((2, Q, CH), jnp.int32),
        pltpu.VMEM_SHARED((N_PAD, DH), jnp.float32),
        pltpu.VMEM_SHARED((N_PAD, DH), jnp.float32),
        pltpu.SemaphoreType.DMA,
        pltpu.SemaphoreType.DMA,
        pltpu.SemaphoreType.DMA,
        pltpu.SemaphoreType.DMA,
        pltpu.SemaphoreType.DMA,
        pltpu.SemaphoreType.DMA,
        pltpu.SemaphoreType.DMA,
        pltpu.SemaphoreType.DMA,
        pltpu.SemaphoreType.DMA,
        pltpu.SemaphoreType.DMA,
    ],
)
def _sc_scatter(x_hbm, srcc_hbm, dstc_hbm, zeros_hbm, out_hbm,
                rows_v, srci_v, dsti_v, agg_sh, x_sh, *sems):
    cid = lax.axis_index("c")
    sid = lax.axis_index("s")
    sem_g = sems[:NBUF]
    sem_s = sems[NBUF:2 * NBUF]
    sem_i0, sem_i1 = sems[2 * NBUF:]

    pltpu.sync_copy(srcc_hbm.at[sid, pl.ds(0, Q)], srci_v.at[0])
    pltpu.sync_copy(dstc_hbm.at[sid, pl.ds(0, Q)], dsti_v.at[0])
    pltpu.sync_copy(x_hbm.at[cid, pl.ds(sid * ZR, ZR)], x_sh.at[pl.ds(sid * ZR, ZR)])
    pltpu.sync_copy(zeros_hbm, agg_sh.at[pl.ds(sid * ZR, ZR)])
    plsc.subcore_barrier()

    xh_hbm = x_hbm.at[cid]

    def gsrc(b):
        return xh_hbm if b in HBM_SLOTS else x_sh

    for grp in range(NGRP):
        buf = grp % 2
        if grp + 1 < NGRP:
            pltpu.async_copy(srcc_hbm.at[sid, pl.ds((grp + 1) * Q, Q)],
                             srci_v.at[1 - buf], sem_i0)
            pltpu.async_copy(dstc_hbm.at[sid, pl.ds((grp + 1) * Q, Q)],
                             dsti_v.at[1 - buf], sem_i1)

        sg = srci_v.at[buf]
        dg = dsti_v.at[buf]
        for b in range(2):
            pltpu.async_copy(gsrc(b).at[sg.at[b]], rows_v.at[b], sem_g[b])

        def group_fn(g, carry, sg=sg, dg=dg):
            for b in range(NBUF):
                k = g * NBUF + b
                b2 = (b + 2) % NBUF

                @pl.when(k + 2 >= NBUF)
                def _():
                    pltpu.make_async_copy(
                        rows_v.at[b2], agg_sh.at[dg.at[k - 2]], sem_s[b2]).wait()

                @pl.when(k + 2 < Q)
                def _():
                    pltpu.async_copy(gsrc(b2).at[sg.at[k + 2]], rows_v.at[b2], sem_g[b2])

                pltpu.make_async_copy(gsrc(b).at[sg.at[k]], rows_v.at[b], sem_g[b]).wait()
                pltpu.async_copy(rows_v.at[b], agg_sh.at[dg.at[k]], sem_s[b], add=True)
            return carry

        lax.fori_loop(0, Q // NBUF, group_fn, 0)
        for k in (Q - 2, Q - 1):
            b = k % NBUF
            pltpu.make_async_copy(rows_v.at[b], agg_sh.at[dg.at[k]], sem_s[b]).wait()
        if grp + 1 < NGRP:
            pltpu.make_async_copy(srcc_hbm.at[sid, pl.ds((grp + 1) * Q, Q)],
                                  srci_v.at[1 - buf], sem_i0).wait()
            pltpu.make_async_copy(dstc_hbm.at[sid, pl.ds((grp + 1) * Q, Q)],
                                  dsti_v.at[1 - buf], sem_i1).wait()

    plsc.subcore_barrier()
    pltpu.sync_copy(agg_sh.at[pl.ds(sid * ZR, ZR)],
                    out_hbm.at[cid, pl.ds(sid * ZR, ZR)])


def _tc_body(x_ref, w_ref, p_ref, o_ref):
    mm = jnp.dot(x_ref[...], w_ref[...], preferred_element_type=jnp.float32)
    agg = jnp.concatenate([p_ref[0], p_ref[1]], axis=1)
    o_ref[...] = jnp.maximum(agg + mm, 0.0)


_BLK = 1000


def _tc_finish(x, w, partials):
    grid = (N // _BLK,)
    return pl.pallas_call(
        _tc_body,
        grid=grid,
        in_specs=[
            pl.BlockSpec((_BLK, D), lambda i: (i, 0)),
            pl.BlockSpec((D, D), lambda i: (0, 0)),
            pl.BlockSpec((NC, _BLK, DH), lambda i: (0, i, 0)),
        ],
        out_specs=pl.BlockSpec((_BLK, D), lambda i: (i, 0)),
        out_shape=jax.ShapeDtypeStruct((N, D), jnp.float32),
    )(x, w, partials)


def kernel(x, edge_index, loop_weight):
    src = edge_index[0].astype(jnp.int32)
    dst = edge_index[1].astype(jnp.int32)
    pad = E_PAD - E
    src_c = jnp.concatenate([src, jnp.zeros((pad,), jnp.int32)]).reshape(NS, CPT, CH)
    dst_c = jnp.concatenate([dst, jnp.full((pad,), N, jnp.int32)]).reshape(NS, CPT, CH)
    zeros = jnp.zeros((ZR, DH), jnp.float32)
    x_split = x.reshape(N, NC, DH).transpose(1, 0, 2)
    x_split = jnp.concatenate(
        [x_split, jnp.zeros((NC, N_PAD - N, DH), jnp.float32)], axis=1)
    partials = _sc_scatter(x_split, src_c, dst_c, zeros)
    return _tc_finish(x, loop_weight, partials)


# speedup vs baseline: 1.4052x; 1.1336x over previous
import functools

import jax
import jax.numpy as jnp
from jax import lax
from jax.experimental import pallas as pl
from jax.experimental.pallas import tpu as pltpu
from jax.experimental.pallas import tpu_sc as plsc

N = 10000
E = 320000
D = 128
DH = D // 2

NC = 2
NS = 16
CH = 128
NBUF = 4
CPT = 160
Q = 32
NGRP = CPT // Q
E_PAD = CPT * NS * CH
N_PAD = 10112
ZR = N_PAD // NS
XTAIL = N - 15 * ZR

_sc_mesh = plsc.VectorSubcoreMesh(core_axis_name="c", subcore_axis_name="s")


@functools.partial(
    pl.kernel,
    out_type=jax.ShapeDtypeStruct((N, D), jnp.float32),
    mesh=_sc_mesh,
    compiler_params=pltpu.CompilerParams(use_tc_tiling_on_sc=False),
    scratch_types=[
        pltpu.VMEM((NBUF, CH, DH), jnp.float32),
        pltpu.VMEM((2, Q, CH), jnp.int32),
        pltpu.VMEM((2, Q, CH), jnp.int32),
        pltpu.VMEM_SHARED((N_PAD, DH), jnp.float32),
        pltpu.VMEM_SHARED((N_PAD, DH), jnp.float32),
        pltpu.SemaphoreType.DMA,
        pltpu.SemaphoreType.DMA,
        pltpu.SemaphoreType.DMA,
        pltpu.SemaphoreType.DMA,
        pltpu.SemaphoreType.DMA,
        pltpu.SemaphoreType.DMA,
        pltpu.SemaphoreType.DMA,
        pltpu.SemaphoreType.DMA,
        pltpu.SemaphoreType.DMA,
        pltpu.SemaphoreType.DMA,
    ],
)
def _sc_scatter(x_hbm, srcc_hbm, dstc_hbm, mm_hbm, out_hbm,
                rows_v, srci_v, dsti_v, agg_sh, x_sh, *sems):
    cid = lax.axis_index("c")
    sid = lax.axis_index("s")
    sem_g = sems[:NBUF]
    sem_s = sems[NBUF:2 * NBUF]
    sem_i0, sem_i1 = sems[2 * NBUF:]

    pltpu.sync_copy(srcc_hbm.at[sid, pl.ds(0, Q)], srci_v.at[0])
    pltpu.sync_copy(dstc_hbm.at[sid, pl.ds(0, Q)], dsti_v.at[0])

    @pl.when(sid < NS - 1)
    def _():
        pltpu.sync_copy(x_hbm.at[pl.ds(sid * ZR, ZR), pl.ds(cid * DH, DH)],
                        x_sh.at[pl.ds(sid * ZR, ZR)])

    @pl.when(sid == NS - 1)
    def _():
        pltpu.sync_copy(x_hbm.at[pl.ds(15 * ZR, XTAIL), pl.ds(cid * DH, DH)],
                        x_sh.at[pl.ds(15 * ZR, XTAIL)])

    pltpu.sync_copy(mm_hbm.at[cid, pl.ds(sid * ZR, ZR)],
                    agg_sh.at[pl.ds(sid * ZR, ZR)])
    plsc.subcore_barrier()

    for grp in range(NGRP):
        buf = grp % 2
        if grp + 1 < NGRP:
            pltpu.async_copy(srcc_hbm.at[sid, pl.ds((grp + 1) * Q, Q)],
                             srci_v.at[1 - buf], sem_i0)
            pltpu.async_copy(dstc_hbm.at[sid, pl.ds((grp + 1) * Q, Q)],
                             dsti_v.at[1 - buf], sem_i1)

        sg = srci_v.at[buf]
        dg = dsti_v.at[buf]
        for b in range(2):
            pltpu.async_copy(x_sh.at[sg.at[b]], rows_v.at[b], sem_g[b])

        def group_fn(g, carry, sg=sg, dg=dg):
            for b in range(NBUF):
                k = g * NBUF + b
                b2 = (b + 2) % NBUF

                @pl.when(k + 2 >= NBUF)
                def _():
                    pltpu.make_async_copy(
                        rows_v.at[b2], agg_sh.at[dg.at[k - 2]], sem_s[b2]).wait()

                @pl.when(k + 2 < Q)
                def _():
                    pltpu.async_copy(x_sh.at[sg.at[k + 2]], rows_v.at[b2], sem_g[b2])

                pltpu.make_async_copy(x_sh.at[sg.at[k]], rows_v.at[b], sem_g[b]).wait()
                pltpu.async_copy(rows_v.at[b], agg_sh.at[dg.at[k]], sem_s[b], add=True)
            return carry

        lax.fori_loop(0, Q // NBUF, group_fn, 0)
        for k in (Q - 2, Q - 1):
            b = k % NBUF
            pltpu.make_async_copy(rows_v.at[b], agg_sh.at[dg.at[k]], sem_s[b]).wait()
        if grp + 1 < NGRP:
            pltpu.make_async_copy(srcc_hbm.at[sid, pl.ds((grp + 1) * Q, Q)],
                                  srci_v.at[1 - buf], sem_i0).wait()
            pltpu.make_async_copy(dstc_hbm.at[sid, pl.ds((grp + 1) * Q, Q)],
                                  dsti_v.at[1 - buf], sem_i1).wait()

    plsc.subcore_barrier()

    def relu_piece(row0, nrows):
        piece = rows_v.at[0]
        pltpu.sync_copy(agg_sh.at[pl.ds(row0, nrows)], piece.at[pl.ds(0, nrows)])

        def rbody(r, carry):
            for c4 in range(DH // 16):
                sl = pl.ds(c4 * 16, 16)
                piece[r, sl] = jnp.maximum(piece[r, sl], 0.0)
            return carry

        lax.fori_loop(0, nrows, rbody, 0)
        pltpu.sync_copy(piece.at[pl.ds(0, nrows)],
                        out_hbm.at[pl.ds(row0, nrows), pl.ds(cid * DH, DH)])

    @pl.when(sid < NS - 1)
    def _():
        base = sid * ZR
        for p, nr in enumerate((128, 128, 128, 128, ZR - 4 * 128)):
            relu_piece(base + p * 128, nr)

    @pl.when(sid == NS - 1)
    def _():
        base = 15 * ZR
        for p, nr in enumerate((128, 128, 128, 128, XTAIL - 4 * 128)):
            relu_piece(base + p * 128, nr)


def _mm_body(x_ref, w_ref, o_ref):
    mm = jnp.dot(x_ref[...], w_ref[...], preferred_element_type=jnp.float32)
    o_ref[0] = mm[:, :DH]
    o_ref[1] = mm[:, DH:]


_BLK = 1000


def _tc_matmul(x, w):
    grid = (N // _BLK,)
    return pl.pallas_call(
        _mm_body,
        grid=grid,
        in_specs=[
            pl.BlockSpec((_BLK, D), lambda i: (i, 0)),
            pl.BlockSpec((D, D), lambda i: (0, 0)),
        ],
        out_specs=pl.BlockSpec((NC, _BLK, DH), lambda i: (0, i, 0)),
        out_shape=jax.ShapeDtypeStruct((NC, N_PAD, DH), jnp.float32),
    )(x, w)


def kernel(x, edge_index, loop_weight):
    src = edge_index[0].astype(jnp.int32)
    dst = edge_index[1].astype(jnp.int32)
    pad = E_PAD - E
    src_c = jnp.concatenate([src, jnp.zeros((pad,), jnp.int32)]).reshape(NS, CPT, CH)
    dst_c = jnp.concatenate([dst, jnp.full((pad,), N, jnp.int32)]).reshape(NS, CPT, CH)
    mm = _tc_matmul(x, loop_weight)
    return _sc_scatter(x, src_c, dst_c, mm)


# speedup vs baseline: 1.4943x; 1.0634x over previous
import functools

import jax
import jax.numpy as jnp
from jax import lax
from jax.experimental import pallas as pl
from jax.experimental.pallas import tpu as pltpu
from jax.experimental.pallas import tpu_sc as plsc

N = 10000
E = 320000
D = 128
DH = D // 2

NC = 2
NS = 16
CH = 128
NBUF = 4
CPT = 160
Q = 32
NGRP = CPT // Q
E_PAD = CPT * NS * CH
N_PAD = 10112
ZR = N_PAD // NS
XTAIL = N - 15 * ZR

_sc_mesh = plsc.VectorSubcoreMesh(core_axis_name="c", subcore_axis_name="s")


@functools.partial(
    pl.kernel,
    out_type=jax.ShapeDtypeStruct((N, D), jnp.float32),
    mesh=_sc_mesh,
    compiler_params=pltpu.CompilerParams(use_tc_tiling_on_sc=False),
    scratch_types=[
        pltpu.VMEM((NBUF, CH, DH), jnp.float32),
        pltpu.VMEM((2, Q, CH), jnp.int32),
        pltpu.VMEM((2, Q, CH), jnp.int32),
        pltpu.VMEM_SHARED((N_PAD, DH), jnp.float32),
        pltpu.VMEM_SHARED((N_PAD, DH), jnp.float32),
        pltpu.SemaphoreType.DMA,
        pltpu.SemaphoreType.DMA,
        pltpu.SemaphoreType.DMA,
        pltpu.SemaphoreType.DMA,
        pltpu.SemaphoreType.DMA,
        pltpu.SemaphoreType.DMA,
        pltpu.SemaphoreType.DMA,
        pltpu.SemaphoreType.DMA,
        pltpu.SemaphoreType.DMA,
        pltpu.SemaphoreType.DMA,
    ],
)
def _sc_scatter(x_hbm, srcc_hbm, dstc_hbm, mm_hbm, out_hbm,
                rows_v, srci_v, dsti_v, agg_sh, x_sh, *sems):
    cid = lax.axis_index("c")
    sid = lax.axis_index("s")
    sem_g = sems[:NBUF]
    sem_s = sems[NBUF:2 * NBUF]
    sem_i0, sem_i1 = sems[2 * NBUF:]

    pltpu.sync_copy(srcc_hbm.at[sid, pl.ds(0, Q)], srci_v.at[0])
    pltpu.sync_copy(dstc_hbm.at[sid, pl.ds(0, Q)], dsti_v.at[0])

    @pl.when(sid < NS - 1)
    def _():
        pltpu.sync_copy(x_hbm.at[pl.ds(sid * ZR, ZR), pl.ds(cid * DH, DH)],
                        x_sh.at[pl.ds(sid * ZR, ZR)])

    @pl.when(sid == NS - 1)
    def _():
        pltpu.sync_copy(x_hbm.at[pl.ds(15 * ZR, XTAIL), pl.ds(cid * DH, DH)],
                        x_sh.at[pl.ds(15 * ZR, XTAIL)])

    pltpu.sync_copy(mm_hbm.at[cid, pl.ds(sid * ZR, ZR)],
                    agg_sh.at[pl.ds(sid * ZR, ZR)])
    plsc.subcore_barrier()

    for grp in range(NGRP):
        buf = grp % 2
        if grp + 1 < NGRP:
            pltpu.async_copy(srcc_hbm.at[sid, pl.ds((grp + 1) * Q, Q)],
                             srci_v.at[1 - buf], sem_i0)
            pltpu.async_copy(dstc_hbm.at[sid, pl.ds((grp + 1) * Q, Q)],
                             dsti_v.at[1 - buf], sem_i1)

        sg = srci_v.at[buf]
        dg = dsti_v.at[buf]
        for b in range(2):
            pltpu.async_copy(x_sh.at[sg.at[b]], rows_v.at[b], sem_g[b])

        def group_fn(g, carry, sg=sg, dg=dg):
            for b in range(NBUF):
                k = g * NBUF + b
                b2 = (b + 2) % NBUF

                @pl.when(k + 2 >= NBUF)
                def _():
                    pltpu.make_async_copy(
                        rows_v.at[b2], agg_sh.at[dg.at[k - 2]], sem_s[b2]).wait()

                @pl.when(k + 2 < Q)
                def _():
                    pltpu.async_copy(x_sh.at[sg.at[k + 2]], rows_v.at[b2], sem_g[b2])

                pltpu.make_async_copy(x_sh.at[sg.at[k]], rows_v.at[b], sem_g[b]).wait()
                pltpu.async_copy(rows_v.at[b], agg_sh.at[dg.at[k]], sem_s[b], add=True)
            return carry

        lax.fori_loop(0, Q // NBUF, group_fn, 0)
        for k in (Q - 2, Q - 1):
            b = k % NBUF
            pltpu.make_async_copy(rows_v.at[b], agg_sh.at[dg.at[k]], sem_s[b]).wait()
        if grp + 1 < NGRP:
            pltpu.make_async_copy(srcc_hbm.at[sid, pl.ds((grp + 1) * Q, Q)],
                                  srci_v.at[1 - buf], sem_i0).wait()
            pltpu.make_async_copy(dstc_hbm.at[sid, pl.ds((grp + 1) * Q, Q)],
                                  dsti_v.at[1 - buf], sem_i1).wait()

    plsc.subcore_barrier()

    def relu_out(base, sizes):
        pieces = []
        r0 = base
        for nr in sizes:
            pieces.append((r0, nr))
            r0 += nr
        np_ = len(pieces)

        def cin(p):
            r0, nr = pieces[p]
            pltpu.async_copy(agg_sh.at[pl.ds(r0, nr)],
                             rows_v.at[p % 3, pl.ds(0, nr)], sem_g[p % 3])

        def cout_desc(p):
            r0, nr = pieces[p]
            return pltpu.make_async_copy(
                rows_v.at[p % 3, pl.ds(0, nr)],
                out_hbm.at[pl.ds(r0, nr), pl.ds(cid * DH, DH)], sem_s[p % 3])

        for p in range(min(3, np_)):
            cin(p)
        for p in range(np_):
            s = p % 3
            if p >= 1 and p + 2 < np_:
                cout_desc(p - 1).wait()
                cin(p + 2)
            r0, nr = pieces[p]
            pltpu.make_async_copy(agg_sh.at[pl.ds(r0, nr)],
                                  rows_v.at[s, pl.ds(0, nr)], sem_g[s]).wait()
            piece = rows_v.at[s]

            def rbody(r, carry):
                for c4 in range(DH // 16):
                    sl = pl.ds(c4 * 16, 16)
                    piece[r, sl] = jnp.maximum(piece[r, sl], 0.0)
                return carry

            lax.fori_loop(0, nr, rbody, 0)
            pltpu.async_copy(rows_v.at[s, pl.ds(0, nr)],
                             out_hbm.at[pl.ds(r0, nr), pl.ds(cid * DH, DH)],
                             sem_s[s])
        for p in range(max(0, np_ - 3), np_):
            cout_desc(p).wait()

    @pl.when(sid < NS - 1)
    def _():
        relu_out(sid * ZR, (128, 128, 128, 128, ZR - 4 * 128))

    @pl.when(sid == NS - 1)
    def _():
        relu_out(15 * ZR, (128, 128, 128, 128, XTAIL - 4 * 128))


def _mm_body(x_ref, w_ref, o_ref):
    mm = jnp.dot(x_ref[...], w_ref[...], preferred_element_type=jnp.float32)
    o_ref[0] = mm[:, :DH]
    o_ref[1] = mm[:, DH:]


_BLK = 1000


def _tc_matmul(x, w):
    grid = (N // _BLK,)
    return pl.pallas_call(
        _mm_body,
        grid=grid,
        in_specs=[
            pl.BlockSpec((_BLK, D), lambda i: (i, 0)),
            pl.BlockSpec((D, D), lambda i: (0, 0)),
        ],
        out_specs=pl.BlockSpec((NC, _BLK, DH), lambda i: (0, i, 0)),
        out_shape=jax.ShapeDtypeStruct((NC, N_PAD, DH), jnp.float32),
    )(x, w)


def kernel(x, edge_index, loop_weight):
    pad = E_PAD - E
    padc = jnp.concatenate(
        [jnp.zeros((1, pad), jnp.int32), jnp.full((1, pad), N, jnp.int32)], axis=0)
    e2 = jnp.concatenate([edge_index.astype(jnp.int32), padc], axis=1)
    src_c = e2[0].reshape(NS, CPT, CH)
    dst_c = e2[1].reshape(NS, CPT, CH)
    mm = _tc_matmul(x, loop_weight)
    return _sc_scatter(x, src_c, dst_c, mm)


# speedup vs baseline: 1.5140x; 1.0132x over previous
import functools

import jax
import jax.numpy as jnp
from jax import lax
from jax.experimental import pallas as pl
from jax.experimental.pallas import tpu as pltpu
from jax.experimental.pallas import tpu_sc as plsc

N = 10000
E = 320000
D = 128
DH = D // 2

NC = 2
NS = 16
CH = 128
NBUF = 4
CPT = 160
Q = 32
NGRP = CPT // Q
E_PAD = CPT * NS * CH
N_PAD = 10112
ZR = N_PAD // NS
XTAIL = N - 15 * ZR

_sc_mesh = plsc.VectorSubcoreMesh(core_axis_name="c", subcore_axis_name="s")


@functools.partial(
    pl.kernel,
    out_type=jax.ShapeDtypeStruct((N, D), jnp.float32),
    mesh=_sc_mesh,
    compiler_params=pltpu.CompilerParams(use_tc_tiling_on_sc=False),
    scratch_types=[
        pltpu.VMEM((NBUF, CH, DH), jnp.float32),
        pltpu.VMEM((2, Q, CH), jnp.int32),
        pltpu.VMEM((2, Q, CH), jnp.int32),
        pltpu.VMEM_SHARED((N_PAD, DH), jnp.float32),
        pltpu.VMEM_SHARED((N_PAD, DH), jnp.float32),
        pltpu.SemaphoreType.DMA,
        pltpu.SemaphoreType.DMA,
        pltpu.SemaphoreType.DMA,
        pltpu.SemaphoreType.DMA,
        pltpu.SemaphoreType.DMA,
        pltpu.SemaphoreType.DMA,
        pltpu.SemaphoreType.DMA,
        pltpu.SemaphoreType.DMA,
        pltpu.SemaphoreType.DMA,
        pltpu.SemaphoreType.DMA,
    ],
)
def _sc_scatter(x_hbm, srcc_hbm, dstc_hbm, mm_hbm, out_hbm,
                rows_v, srci_v, dsti_v, agg_sh, x_sh, *sems):
    cid = lax.axis_index("c")
    sid = lax.axis_index("s")
    sem_g = sems[:NBUF]
    sem_s = sems[NBUF:2 * NBUF]
    sem_i0, sem_i1 = sems[2 * NBUF:]

    pltpu.async_copy(srcc_hbm.at[sid, pl.ds(0, Q)], srci_v.at[0], sem_i0)
    pltpu.async_copy(dstc_hbm.at[sid, pl.ds(0, Q)], dsti_v.at[0], sem_i1)

    @pl.when(sid < NS - 1)
    def _():
        pltpu.async_copy(x_hbm.at[pl.ds(sid * ZR, ZR), pl.ds(cid * DH, DH)],
                         x_sh.at[pl.ds(sid * ZR, ZR)], sem_g[0])

    @pl.when(sid == NS - 1)
    def _():
        pltpu.async_copy(x_hbm.at[pl.ds(15 * ZR, XTAIL), pl.ds(cid * DH, DH)],
                         x_sh.at[pl.ds(15 * ZR, XTAIL)], sem_g[0])

    pltpu.async_copy(mm_hbm.at[cid, pl.ds(sid * ZR, ZR)],
                     agg_sh.at[pl.ds(sid * ZR, ZR)], sem_s[0])

    @pl.when(sid < NS - 1)
    def _():
        pltpu.make_async_copy(x_hbm.at[pl.ds(sid * ZR, ZR), pl.ds(cid * DH, DH)],
                              x_sh.at[pl.ds(sid * ZR, ZR)], sem_g[0]).wait()

    @pl.when(sid == NS - 1)
    def _():
        pltpu.make_async_copy(x_hbm.at[pl.ds(15 * ZR, XTAIL), pl.ds(cid * DH, DH)],
                              x_sh.at[pl.ds(15 * ZR, XTAIL)], sem_g[0]).wait()

    pltpu.make_async_copy(mm_hbm.at[cid, pl.ds(sid * ZR, ZR)],
                          agg_sh.at[pl.ds(sid * ZR, ZR)], sem_s[0]).wait()
    pltpu.make_async_copy(srcc_hbm.at[sid, pl.ds(0, Q)], srci_v.at[0], sem_i0).wait()
    pltpu.make_async_copy(dstc_hbm.at[sid, pl.ds(0, Q)], dsti_v.at[0], sem_i1).wait()
    plsc.subcore_barrier()

    for grp in range(NGRP):
        buf = grp % 2
        if grp + 1 < NGRP:
            pltpu.async_copy(srcc_hbm.at[sid, pl.ds((grp + 1) * Q, Q)],
                             srci_v.at[1 - buf], sem_i0)
            pltpu.async_copy(dstc_hbm.at[sid, pl.ds((grp + 1) * Q, Q)],
                             dsti_v.at[1 - buf], sem_i1)

        sg = srci_v.at[buf]
        dg = dsti_v.at[buf]
        for b in range(2):
            pltpu.async_copy(x_sh.at[sg.at[b]], rows_v.at[b], sem_g[b])

        def group_fn(g, carry, sg=sg, dg=dg):
            for b in range(NBUF):
                k = g * NBUF + b
                b2 = (b + 2) % NBUF

                @pl.when(k + 2 >= NBUF)
                def _():
                    pltpu.make_async_copy(
                        rows_v.at[b2], agg_sh.at[dg.at[k - 2]], sem_s[b2]).wait()

                @pl.when(k + 2 < Q)
                def _():
                    pltpu.async_copy(x_sh.at[sg.at[k + 2]], rows_v.at[b2], sem_g[b2])

                pltpu.make_async_copy(x_sh.at[sg.at[k]], rows_v.at[b], sem_g[b]).wait()
                pltpu.async_copy(rows_v.at[b], agg_sh.at[dg.at[k]], sem_s[b], add=True)
            return carry

        lax.fori_loop(0, Q // NBUF, group_fn, 0)
        for k in (Q - 2, Q - 1):
            b = k % NBUF
            pltpu.make_async_copy(rows_v.at[b], agg_sh.at[dg.at[k]], sem_s[b]).wait()
        if grp + 1 < NGRP:
            pltpu.make_async_copy(srcc_hbm.at[sid, pl.ds((grp + 1) * Q, Q)],
                                  srci_v.at[1 - buf], sem_i0).wait()
            pltpu.make_async_copy(dstc_hbm.at[sid, pl.ds((grp + 1) * Q, Q)],
                                  dsti_v.at[1 - buf], sem_i1).wait()

    plsc.subcore_barrier()

    def relu_out(base, sizes):
        pieces = []
        r0 = base
        for nr in sizes:
            pieces.append((r0, nr))
            r0 += nr
        np_ = len(pieces)

        def cin(p):
            r0, nr = pieces[p]
            pltpu.async_copy(agg_sh.at[pl.ds(r0, nr)],
                             rows_v.at[p % 3, pl.ds(0, nr)], sem_g[p % 3])

        def cout_desc(p):
            r0, nr = pieces[p]
            return pltpu.make_async_copy(
                rows_v.at[p % 3, pl.ds(0, nr)],
                out_hbm.at[pl.ds(r0, nr), pl.ds(cid * DH, DH)], sem_s[p % 3])

        for p in range(min(3, np_)):
            cin(p)
        for p in range(np_):
            s = p % 3
            if p >= 1 and p + 2 < np_:
                cout_desc(p - 1).wait()
                cin(p + 2)
            r0, nr = pieces[p]
            pltpu.make_async_copy(agg_sh.at[pl.ds(r0, nr)],
                                  rows_v.at[s, pl.ds(0, nr)], sem_g[s]).wait()
            piece = rows_v.at[s]

            def rbody(r, carry):
                for c4 in range(DH // 16):
                    sl = pl.ds(c4 * 16, 16)
                    piece[r, sl] = jnp.maximum(piece[r, sl], 0.0)
                return carry

            lax.fori_loop(0, nr, rbody, 0)
            pltpu.async_copy(rows_v.at[s, pl.ds(0, nr)],
                             out_hbm.at[pl.ds(r0, nr), pl.ds(cid * DH, DH)],
                             sem_s[s])
        for p in range(max(0, np_ - 3), np_):
            cout_desc(p).wait()

    @pl.when(sid < NS - 1)
    def _():
        relu_out(sid * ZR, (128, 128, 128, 128, ZR - 4 * 128))

    @pl.when(sid == NS - 1)
    def _():
        relu_out(15 * ZR, (128, 128, 128, 128, XTAIL - 4 * 128))


def _mm_body(x_ref, w_ref, o_ref):
    mm = jnp.dot(x_ref[...], w_ref[...], preferred_element_type=jnp.float32)
    o_ref[0] = mm[:, :DH]
    o_ref[1] = mm[:, DH:]


_BLK = 1000


def _tc_matmul(x, w):
    grid = (N // _BLK,)
    return pl.pallas_call(
        _mm_body,
        grid=grid,
        in_specs=[
            pl.BlockSpec((_BLK, D), lambda i: (i, 0)),
            pl.BlockSpec((D, D), lambda i: (0, 0)),
        ],
        out_specs=pl.BlockSpec((NC, _BLK, DH), lambda i: (0, i, 0)),
        out_shape=jax.ShapeDtypeStruct((NC, N_PAD, DH), jnp.float32),
    )(x, w)


def kernel(x, edge_index, loop_weight):
    pad = E_PAD - E
    padc = jnp.concatenate(
        [jnp.zeros((1, pad), jnp.int32), jnp.full((1, pad), N, jnp.int32)], axis=0)
    e2 = jnp.concatenate([edge_index.astype(jnp.int32), padc], axis=1)
    src_c = e2[0].reshape(NS, CPT, CH)
    dst_c = e2[1].reshape(NS, CPT, CH)
    mm = _tc_matmul(x, loop_weight)
    return _sc_scatter(x, src_c, dst_c, mm)


# speedup vs baseline: 1.5672x; 1.0351x over previous
import functools

import jax
import jax.numpy as jnp
from jax import lax
from jax.experimental import pallas as pl
from jax.experimental.pallas import tpu as pltpu
from jax.experimental.pallas import tpu_sc as plsc

N = 10000
E = 320000
D = 128
DH = D // 2

NC = 2
NS = 16
CH = 128
NBUF = 4
CPT = 160
Q = 32
NGRP = CPT // Q
E_PAD = CPT * NS * CH
N_PAD = 10112
ZR = N_PAD // NS
XTAIL = N - 15 * ZR

_sc_mesh = plsc.VectorSubcoreMesh(core_axis_name="c", subcore_axis_name="s")


@functools.partial(
    pl.kernel,
    out_type=jax.ShapeDtypeStruct((N, D), jnp.float32),
    mesh=_sc_mesh,
    compiler_params=pltpu.CompilerParams(use_tc_tiling_on_sc=False),
    scratch_types=[
        pltpu.VMEM((NBUF, CH, DH), jnp.float32),
        pltpu.VMEM((2, Q, CH), jnp.int32),
        pltpu.VMEM((2, Q, CH), jnp.int32),
        pltpu.VMEM_SHARED((N_PAD, DH), jnp.float32),
        pltpu.VMEM_SHARED((N_PAD, DH), jnp.float32),
        pltpu.SemaphoreType.DMA,
        pltpu.SemaphoreType.DMA,
        pltpu.SemaphoreType.DMA,
        pltpu.SemaphoreType.DMA,
        pltpu.SemaphoreType.DMA,
        pltpu.SemaphoreType.DMA,
        pltpu.SemaphoreType.DMA,
        pltpu.SemaphoreType.DMA,
        pltpu.SemaphoreType.DMA,
        pltpu.SemaphoreType.DMA,
    ],
)
def _sc_scatter(x_hbm, srcc_hbm, dstc_hbm, mm_hbm, out_hbm,
                rows_v, srci_v, dsti_v, agg_sh, x_sh, *sems):
    cid = lax.axis_index("c")
    sid = lax.axis_index("s")
    sem_g = sems[:NBUF]
    sem_s = sems[NBUF:2 * NBUF]
    sem_i0, sem_i1 = sems[2 * NBUF:]

    pltpu.async_copy(srcc_hbm.at[sid, pl.ds(0, Q)], srci_v.at[0], sem_i0)
    pltpu.async_copy(dstc_hbm.at[sid, pl.ds(0, Q)], dsti_v.at[0], sem_i1)

    @pl.when(sid < NS - 1)
    def _():
        pltpu.async_copy(x_hbm.at[pl.ds(sid * ZR, ZR), pl.ds(cid * DH, DH)],
                         x_sh.at[pl.ds(sid * ZR, ZR)], sem_g[0])

    @pl.when(sid == NS - 1)
    def _():
        pltpu.async_copy(x_hbm.at[pl.ds(15 * ZR, XTAIL), pl.ds(cid * DH, DH)],
                         x_sh.at[pl.ds(15 * ZR, XTAIL)], sem_g[0])

    pltpu.async_copy(mm_hbm.at[cid, pl.ds(sid * ZR, ZR)],
                     agg_sh.at[pl.ds(sid * ZR, ZR)], sem_s[0])

    @pl.when(sid < NS - 1)
    def _():
        pltpu.make_async_copy(x_hbm.at[pl.ds(sid * ZR, ZR), pl.ds(cid * DH, DH)],
                              x_sh.at[pl.ds(sid * ZR, ZR)], sem_g[0]).wait()

    @pl.when(sid == NS - 1)
    def _():
        pltpu.make_async_copy(x_hbm.at[pl.ds(15 * ZR, XTAIL), pl.ds(cid * DH, DH)],
                              x_sh.at[pl.ds(15 * ZR, XTAIL)], sem_g[0]).wait()

    pltpu.make_async_copy(mm_hbm.at[cid, pl.ds(sid * ZR, ZR)],
                          agg_sh.at[pl.ds(sid * ZR, ZR)], sem_s[0]).wait()
    pltpu.make_async_copy(srcc_hbm.at[sid, pl.ds(0, Q)], srci_v.at[0], sem_i0).wait()
    pltpu.make_async_copy(dstc_hbm.at[sid, pl.ds(0, Q)], dsti_v.at[0], sem_i1).wait()
    plsc.subcore_barrier()

    def sidx(j):
        return srci_v.at[lax.rem(lax.div(j, Q), 2), lax.rem(j, Q)]

    def didx(j):
        return dsti_v.at[lax.rem(lax.div(j, Q), 2), lax.rem(j, Q)]

    for b in range(2):
        pltpu.async_copy(x_sh.at[sidx(b)], rows_v.at[b], sem_g[b])

    def group_fn(g, carry):
        for b in range(NBUF):
            j = g * NBUF + b
            b2 = (b + 2) % NBUF

            if b == 2:
                @pl.when(lax.rem(j, Q) == 2)
                def _():
                    nxt = (lax.div(j, Q) + 1) * Q
                    nbuf = lax.rem(lax.div(j, Q) + 1, 2)

                    @pl.when(nxt < CPT)
                    def _():
                        pltpu.async_copy(
                            srcc_hbm.at[sid, pl.ds(pl.multiple_of(nxt, Q), Q)],
                            srci_v.at[nbuf], sem_i0)
                        pltpu.async_copy(
                            dstc_hbm.at[sid, pl.ds(pl.multiple_of(nxt, Q), Q)],
                            dsti_v.at[nbuf], sem_i1)

                @pl.when(lax.rem(j, Q) == Q - 2)
                def _():
                    nxt = (lax.div(j, Q) + 1) * Q
                    nbuf = lax.rem(lax.div(j, Q) + 1, 2)

                    @pl.when(nxt < CPT)
                    def _():
                        pltpu.make_async_copy(
                            srcc_hbm.at[sid, pl.ds(pl.multiple_of(nxt, Q), Q)],
                            srci_v.at[nbuf], sem_i0).wait()
                        pltpu.make_async_copy(
                            dstc_hbm.at[sid, pl.ds(pl.multiple_of(nxt, Q), Q)],
                            dsti_v.at[nbuf], sem_i1).wait()

            @pl.when(j + 2 >= NBUF)
            def _():
                pltpu.make_async_copy(
                    rows_v.at[b2], agg_sh.at[didx(j - 2)], sem_s[b2]).wait()

            @pl.when(j + 2 < CPT)
            def _():
                pltpu.async_copy(x_sh.at[sidx(j + 2)], rows_v.at[b2], sem_g[b2])

            pltpu.make_async_copy(x_sh.at[sidx(j)], rows_v.at[b], sem_g[b]).wait()
            pltpu.async_copy(rows_v.at[b], agg_sh.at[didx(j)], sem_s[b], add=True)
        return carry

    lax.fori_loop(0, CPT // NBUF, group_fn, 0)
    for j in (CPT - 2, CPT - 1):
        b = j % NBUF
        pltpu.make_async_copy(rows_v.at[b], agg_sh.at[didx(j)], sem_s[b]).wait()

    plsc.subcore_barrier()

    def relu_out(base, sizes):
        pieces = []
        r0 = base
        for nr in sizes:
            pieces.append((r0, nr))
            r0 += nr
        np_ = len(pieces)

        def cin(p):
            r0, nr = pieces[p]
            pltpu.async_copy(agg_sh.at[pl.ds(r0, nr)],
                             rows_v.at[p % 3, pl.ds(0, nr)], sem_g[p % 3])

        def cout_desc(p):
            r0, nr = pieces[p]
            return pltpu.make_async_copy(
                rows_v.at[p % 3, pl.ds(0, nr)],
                out_hbm.at[pl.ds(r0, nr), pl.ds(cid * DH, DH)], sem_s[p % 3])

        for p in range(min(3, np_)):
            cin(p)
        for p in range(np_):
            s = p % 3
            if p >= 1 and p + 2 < np_:
                cout_desc(p - 1).wait()
                cin(p + 2)
            r0, nr = pieces[p]
            pltpu.make_async_copy(agg_sh.at[pl.ds(r0, nr)],
                                  rows_v.at[s, pl.ds(0, nr)], sem_g[s]).wait()
            piece = rows_v.at[s]

            def rbody(r, carry):
                for c4 in range(DH // 16):
                    sl = pl.ds(c4 * 16, 16)
                    piece[r, sl] = jnp.maximum(piece[r, sl], 0.0)
                return carry

            lax.fori_loop(0, nr, rbody, 0)
            pltpu.async_copy(rows_v.at[s, pl.ds(0, nr)],
                             out_hbm.at[pl.ds(r0, nr), pl.ds(cid * DH, DH)],
                             sem_s[s])
        for p in range(max(0, np_ - 3), np_):
            cout_desc(p).wait()

    @pl.when(sid < NS - 1)
    def _():
        relu_out(sid * ZR, (128, 128, 128, 128, ZR - 4 * 128))

    @pl.when(sid == NS - 1)
    def _():
        relu_out(15 * ZR, (128, 128, 128, 128, XTAIL - 4 * 128))


def _mm_body(x_ref, w_ref, o_ref):
    mm = jnp.dot(x_ref[...], w_ref[...], preferred_element_type=jnp.float32)
    o_ref[0] = mm[:, :DH]
    o_ref[1] = mm[:, DH:]


_BLK = 1000


def _tc_matmul(x, w):
    grid = (N // _BLK,)
    return pl.pallas_call(
        _mm_body,
        grid=grid,
        in_specs=[
            pl.BlockSpec((_BLK, D), lambda i: (i, 0)),
            pl.BlockSpec((D, D), lambda i: (0, 0)),
        ],
        out_specs=pl.BlockSpec((NC, _BLK, DH), lambda i: (0, i, 0)),
        out_shape=jax.ShapeDtypeStruct((NC, N_PAD, DH), jnp.float32),
    )(x, w)


def kernel(x, edge_index, loop_weight):
    pad = E_PAD - E
    padc = jnp.concatenate(
        [jnp.zeros((1, pad), jnp.int32), jnp.full((1, pad), N, jnp.int32)], axis=0)
    e2 = jnp.concatenate([edge_index.astype(jnp.int32), padc], axis=1)
    src_c = e2[0].reshape(NS, CPT, CH)
    dst_c = e2[1].reshape(NS, CPT, CH)
    mm = _tc_matmul(x, loop_weight)
    return _sc_scatter(x, src_c, dst_c, mm)
